# Initial kernel scaffold; baseline (speedup 1.0000x reference)
import jax
import jax.numpy as jnp
from jax.experimental import pallas as pl


def kernel(x, A):
    raise NotImplementedError("write your pallas kernel here")


# speedup vs baseline: 1.7963x; 1.7963x over previous
import functools

import jax
import jax.numpy as jnp
from jax import lax
from jax.experimental import pallas as pl
from jax.experimental.pallas import tpu as pltpu
from jax.experimental.pallas import tpu_sc as plsc

VOCAB = 1000000
EMBED = 64
BATCH = 16384
HIST = 50

B = BATCH * HIST
NC = 2
NS = 16
NW = NC * NS
BPW = B // NW

SUB = 128
CHUNK = 512
NSUB = CHUNK // SUB
NBLK = BPW // CHUNK

_mesh = plsc.VectorSubcoreMesh(core_axis_name="c", subcore_axis_name="s")


@functools.partial(
    pl.kernel,
    mesh=_mesh,
    out_type=jax.ShapeDtypeStruct((B, EMBED), jnp.float32),
    compiler_params=pltpu.CompilerParams(use_tc_tiling_on_sc=False),
    scratch_types=[
        pltpu.VMEM((NSUB, SUB), jnp.int32),
        pltpu.VMEM((CHUNK, EMBED), jnp.float32),
        pltpu.SemaphoreType.DMA,
    ],
)
def _emb_lookup(x_hbm, a_hbm, out_hbm, idx_v, rows_v, gsem):
    wid = lax.axis_index("s") * NC + lax.axis_index("c")
    row0 = wid * (BPW // SUB)

    def blk(i, carry):
        pltpu.sync_copy(x_hbm.at[pl.ds(row0 + i * NSUB, NSUB)], idx_v)
        copies = [
            pltpu.async_copy(
                a_hbm.at[idx_v.at[j]],
                rows_v.at[pl.ds(j * SUB, SUB)],
                gsem,
            )
            for j in range(NSUB)
        ]
        for c in copies:
            c.wait()
        off = wid * BPW + i * CHUNK
        pltpu.sync_copy(rows_v, out_hbm.at[pl.ds(off, CHUNK)])
        return carry

    lax.fori_loop(0, NBLK, blk, 0)


def kernel(x, A):
    xf = x.reshape(-1).astype(jnp.int32).reshape(B // SUB, SUB)
    out = _emb_lookup(xf, A)
    return out.reshape(BATCH, HIST, EMBED)


# speedup vs baseline: 1.8728x; 1.0426x over previous
import functools

import jax
import jax.numpy as jnp
from jax import lax
from jax.experimental import pallas as pl
from jax.experimental.pallas import tpu as pltpu
from jax.experimental.pallas import tpu_sc as plsc

VOCAB = 1000000
EMBED = 64
BATCH = 16384
HIST = 50

B = BATCH * HIST
NC = 2
NS = 16
NW = NC * NS
BPW = B // NW

SUB = 128
CHUNK = 512
NSUB = CHUNK // SUB
NBLK = BPW // CHUNK
NBUF = 2

_mesh = plsc.VectorSubcoreMesh(core_axis_name="c", subcore_axis_name="s")


@functools.partial(
    pl.kernel,
    mesh=_mesh,
    out_type=jax.ShapeDtypeStruct((B, EMBED), jnp.float32),
    compiler_params=pltpu.CompilerParams(use_tc_tiling_on_sc=False),
    scratch_types=[
        pltpu.VMEM((NBUF, NSUB, SUB), jnp.int32),
        pltpu.VMEM((NBUF, CHUNK, EMBED), jnp.float32),
        pltpu.SemaphoreType.DMA((NBUF,)),
        pltpu.SemaphoreType.DMA,
        pltpu.SemaphoreType.DMA((NBUF,)),
    ],
)
def _emb_lookup(x_hbm, a_hbm, out_hbm, idx_v, rows_v, isem, gsem, wsem):
    wid = lax.axis_index("s") * NC + lax.axis_index("c")
    irow0 = wid * (BPW // SUB)
    obase = wid * BPW

    def start_idx(i, b):
        pltpu.async_copy(x_hbm.at[pl.ds(irow0 + i * NSUB, NSUB)], idx_v.at[b], isem.at[b])

    def drain_idx(b):
        pltpu.make_async_copy(x_hbm.at[pl.ds(0, NSUB)], idx_v.at[b], isem.at[b]).wait()

    def drain_write(b):
        pltpu.make_async_copy(rows_v.at[b], out_hbm.at[pl.ds(0, CHUNK)], wsem.at[b]).wait()

    for b in range(NBUF):
        start_idx(b, b)

    def step(i0, carry):
        for b in range(NBUF):
            i = i0 + b
            drain_idx(b)
            @pl.when(i >= NBUF)
            def _():
                drain_write(b)
            copies = [
                pltpu.async_copy(
                    a_hbm.at[idx_v.at[b].at[j]],
                    rows_v.at[b].at[pl.ds(j * SUB, SUB)],
                    gsem,
                )
                for j in range(NSUB)
            ]
            for c in copies:
                c.wait()
            @pl.when(i + NBUF < NBLK)
            def _():
                start_idx(i + NBUF, b)
            pltpu.async_copy(rows_v.at[b], out_hbm.at[pl.ds(obase + i * CHUNK, CHUNK)], wsem.at[b])
        return carry

    lax.fori_loop(0, NBLK // NBUF, lambda k, c: step(k * NBUF, c), 0)

    for b in range(NBUF):
        drain_write(b)


def kernel(x, A):
    xf = x.reshape(-1).astype(jnp.int32).reshape(B // SUB, SUB)
    out = _emb_lookup(xf, A)
    return out.reshape(BATCH, HIST, EMBED)
